# baseline (speedup 1.0000x reference)
import functools

import jax
import jax.numpy as jnp
from jax import lax
from jax.experimental import pallas as pl
from jax.experimental.pallas import tpu as pltpu
from jax.experimental.pallas import tpu_sc as plsc

DIM = 64
LANES = 16
NW = 32
B = 16384
B_PER_W = B // NW
CH = B // NW // 8
NCH = B_PER_W // CH
NPASS = 8
ROWS_P = B_PER_W // NPASS
GROUPS_P = ROWS_P // LANES
GROUPS_W = B_PER_W // LANES
ENT_TOT = 1000000
LB = 32768
ENT_BLKS = (ENT_TOT + LB - 1) // LB
ENT_P = ENT_BLKS * LB // 2
REL_P = 500


def _pair_table(entT):
    def body(entT_ref, out_ref):
        blk = entT_ref[...]
        stacked = jnp.concatenate(
            [blk[:, :LB // 2], blk[:, LB // 2:]], axis=0)
        out_ref[...] = jnp.transpose(stacked)

    return pl.pallas_call(
        body,
        grid=(ENT_BLKS,),
        in_specs=[pl.BlockSpec((DIM, LB), lambda g: (0, g))],
        out_specs=pl.BlockSpec((LB // 2, 2 * DIM), lambda g: (g, 0)),
        out_shape=jax.ShapeDtypeStruct((ENT_P, 2 * DIM), jnp.float32),
    )(entT)


def _make_kernel():
    @functools.partial(
        pl.kernel,
        mesh=plsc.VectorSubcoreMesh(core_axis_name="c", subcore_axis_name="s"),
        out_type=jax.ShapeDtypeStruct((B,), jnp.float32),
        compiler_params=pltpu.CompilerParams(use_tc_tiling_on_sc=True),
        scratch_types=[
            pltpu.VMEM((NCH, CH), jnp.int32),
            pltpu.VMEM((NCH, CH), jnp.int32),
            pltpu.VMEM((NCH, CH), jnp.int32),
            pltpu.VMEM((GROUPS_W, LANES), jnp.int32),
            pltpu.VMEM((2, ROWS_P, 2 * DIM), jnp.float32),
            pltpu.VMEM((2, ROWS_P, 2 * DIM), jnp.float32),
            pltpu.VMEM((2, ROWS_P, 2 * DIM), jnp.float32),
            pltpu.VMEM((B_PER_W,), jnp.float32),
            pltpu.SemaphoreType.DMA((2,)),
        ],
    )
    def distmult(h_idx_hbm, t_idx_hbm, r_idx_hbm, par_hbm,
                 ent_hbm, rel_hbm, out_hbm,
                 hidx_v, tidx_v, ridx_v, par_v,
                 h_v, t_v, r_v, out_v, sem):
        wid = lax.axis_index("s") * 2 + lax.axis_index("c")
        row0 = wid * NCH
        pltpu.sync_copy(h_idx_hbm.at[pl.ds(row0, NCH)], hidx_v)
        pltpu.sync_copy(t_idx_hbm.at[pl.ds(row0, NCH)], tidx_v)
        pltpu.sync_copy(r_idx_hbm.at[pl.ds(row0, NCH)], ridx_v)
        pltpu.sync_copy(par_hbm.at[wid], par_v)

        lane = jnp.arange(LANES, dtype=jnp.int32)
        perms = [(lane ^ (1 << b)).reshape(LANES, 1) for b in range(4)]
        dnums = lax.GatherDimensionNumbers(
            offset_dims=(), collapsed_slice_dims=(0,), start_index_map=(0,))

        def shuffle(x, pm):
            return lax.gather(x, pm, dnums, (1,),
                              mode=lax.GatherScatterMode.PROMISE_IN_BOUNDS)

        def fire(p):
            s = p % 2
            return [
                pltpu.async_copy(ent_hbm.at[hidx_v.at[p]], h_v.at[s], sem.at[s]),
                pltpu.async_copy(ent_hbm.at[tidx_v.at[p]], t_v.at[s], sem.at[s]),
                pltpu.async_copy(rel_hbm.at[ridx_v.at[p]], r_v.at[s], sem.at[s]),
            ]

        pending = {0: fire(0)}
        for p in range(NPASS):
            if p + 1 < NPASS:
                pending[p + 1] = fire(p + 1)
            for cp in pending.pop(p):
                cp.wait()
            s = p % 2

            def group(g, _):
                gg = p * GROUPS_P + g
                pv = par_v[gg, pl.ds(0, LANES)]
                wh = pv[0]
                wt = pv[1]
                wr = pv[2]
                acc = jnp.zeros((LANES,), jnp.float32)
                for rr in range(LANES):
                    r_i = g * LANES + rr
                    oh = ((wh >> rr) & 1) * DIM
                    ot = ((wt >> rr) & 1) * DIM
                    orr = ((wr >> rr) & 1) * DIM
                    acc4 = None
                    for k in range(DIM // LANES):
                        ph = h_v[s, r_i, pl.ds(oh + k * LANES, LANES)]
                        pt = t_v[s, r_i, pl.ds(ot + k * LANES, LANES)]
                        pr = r_v[s, r_i, pl.ds(orr + k * LANES, LANES)]
                        prod = ph * pt * pr
                        acc4 = prod if acc4 is None else acc4 + prod
                    for pm in perms:
                        acc4 = acc4 + shuffle(acc4, pm)
                    acc = jnp.where(lane == rr, acc4, acc)
                out_v[pl.ds(p * ROWS_P + g * LANES, LANES)] = acc
                return 0

            lax.fori_loop(0, GROUPS_P, group, 0)

        pltpu.sync_copy(out_v, out_hbm.at[pl.ds(wid * B_PER_W, B_PER_W)])

    return distmult


_distmult = _make_kernel()


def kernel(batch_h, batch_t, batch_r, ent_embeddings, rel_embeddings):
    bh = batch_h.astype(jnp.int32)
    bt = batch_t.astype(jnp.int32)
    br = batch_r.astype(jnp.int32)
    half = LB // 2
    hp = (((bh // LB) * half) | (bh & (half - 1))).reshape(NW * NCH, CH)
    tp = (((bt // LB) * half) | (bt & (half - 1))).reshape(NW * NCH, CH)
    rp = (br >> 1).reshape(NW * NCH, CH)
    shifts = jnp.arange(LANES, dtype=jnp.int32)
    ph = jnp.sum((((bh // half) & 1).reshape(NW, GROUPS_W, LANES)) << shifts,
                 axis=-1)
    pt = jnp.sum((((bt // half) & 1).reshape(NW, GROUPS_W, LANES)) << shifts,
                 axis=-1)
    pr = jnp.sum(((br & 1).reshape(NW, GROUPS_W, LANES)) << shifts, axis=-1)
    par = jnp.stack([ph, pt, pr], axis=-1).astype(jnp.int32)
    par = jnp.pad(par, ((0, 0), (0, 0), (0, LANES - 3)))
    ent2 = _pair_table(ent_embeddings.T)
    rel2 = rel_embeddings.reshape(REL_P, 2 * DIM)
    return _distmult(hp, tp, rp, par, ent2, rel2)


# speedup vs baseline: 1.0100x; 1.0100x over previous
import functools

import jax
import jax.numpy as jnp
from jax import lax
from jax.experimental import pallas as pl
from jax.experimental.pallas import tpu as pltpu
from jax.experimental.pallas import tpu_sc as plsc

DIM = 64
LANES = 16
NW = 32
B = 16384
B_PER_W = B // NW
CH = 128
NCH = B_PER_W // CH
NPASS = 4
ROWS_P = B_PER_W // NPASS
GROUPS_P = ROWS_P // LANES
GROUPS_W = B_PER_W // LANES
ENT_TOT = 1000000
LB = 32768
ENT_BLKS = (ENT_TOT + LB - 1) // LB
ENT_P = ENT_BLKS * LB // 2
REL_P = 500


def _pair_table(entT):
    def body(entT_ref, out_ref):
        blk = entT_ref[...]
        stacked = jnp.concatenate(
            [blk[:, :LB // 2], blk[:, LB // 2:]], axis=0)
        out_ref[...] = jnp.transpose(stacked)

    return pl.pallas_call(
        body,
        grid=(ENT_BLKS,),
        in_specs=[pl.BlockSpec((DIM, LB), lambda g: (0, g))],
        out_specs=pl.BlockSpec((LB // 2, 2 * DIM), lambda g: (g, 0)),
        out_shape=jax.ShapeDtypeStruct((ENT_P, 2 * DIM), jnp.float32),
    )(entT)


def _make_kernel():
    @functools.partial(
        pl.kernel,
        mesh=plsc.VectorSubcoreMesh(core_axis_name="c", subcore_axis_name="s"),
        out_type=jax.ShapeDtypeStruct((B,), jnp.float32),
        compiler_params=pltpu.CompilerParams(use_tc_tiling_on_sc=True),
        scratch_types=[
            pltpu.VMEM((NCH, CH), jnp.int32),
            pltpu.VMEM((NCH, CH), jnp.int32),
            pltpu.VMEM((NCH, CH), jnp.int32),
            pltpu.VMEM((GROUPS_W, LANES), jnp.int32),
            pltpu.VMEM((2, ROWS_P, 2 * DIM), jnp.float32),
            pltpu.VMEM((2, ROWS_P, 2 * DIM), jnp.float32),
            pltpu.VMEM((2, ROWS_P, 2 * DIM), jnp.float32),
            pltpu.VMEM((B_PER_W,), jnp.float32),
            pltpu.SemaphoreType.DMA((2,)),
        ],
    )
    def distmult(h_idx_hbm, t_idx_hbm, r_idx_hbm, par_hbm,
                 ent_hbm, rel_hbm, out_hbm,
                 hidx_v, tidx_v, ridx_v, par_v,
                 h_v, t_v, r_v, out_v, sem):
        wid = lax.axis_index("s") * 2 + lax.axis_index("c")
        row0 = wid * NCH
        pltpu.sync_copy(h_idx_hbm.at[pl.ds(row0, NCH)], hidx_v)
        pltpu.sync_copy(t_idx_hbm.at[pl.ds(row0, NCH)], tidx_v)
        pltpu.sync_copy(r_idx_hbm.at[pl.ds(row0, NCH)], ridx_v)
        pltpu.sync_copy(par_hbm.at[wid], par_v)

        lane = jnp.arange(LANES, dtype=jnp.int32)
        perms = [(lane ^ (1 << b)).reshape(LANES, 1) for b in range(4)]
        dnums = lax.GatherDimensionNumbers(
            offset_dims=(), collapsed_slice_dims=(0,), start_index_map=(0,))

        def shuffle(x, pm):
            return lax.gather(x, pm, dnums, (1,),
                              mode=lax.GatherScatterMode.PROMISE_IN_BOUNDS)

        def fire(p):
            s = p % 2
            return [
                pltpu.async_copy(ent_hbm.at[hidx_v.at[p]], h_v.at[s], sem.at[s]),
                pltpu.async_copy(ent_hbm.at[tidx_v.at[p]], t_v.at[s], sem.at[s]),
                pltpu.async_copy(rel_hbm.at[ridx_v.at[p]], r_v.at[s], sem.at[s]),
            ]

        pending = {0: fire(0)}
        for p in range(NPASS):
            if p + 1 < NPASS:
                pending[p + 1] = fire(p + 1)
            for cp in pending.pop(p):
                cp.wait()
            s = p % 2

            def group(g, _):
                gg = p * GROUPS_P + g
                pv = par_v[gg, pl.ds(0, LANES)]
                wh = pv[0]
                wt = pv[1]
                wr = pv[2]
                acc = jnp.zeros((LANES,), jnp.float32)
                for rr in range(LANES):
                    r_i = g * LANES + rr
                    oh = ((wh >> rr) & 1) * DIM
                    ot = ((wt >> rr) & 1) * DIM
                    orr = ((wr >> rr) & 1) * DIM
                    acc4 = None
                    for k in range(DIM // LANES):
                        ph = h_v[s, r_i, pl.ds(oh + k * LANES, LANES)]
                        pt = t_v[s, r_i, pl.ds(ot + k * LANES, LANES)]
                        pr = r_v[s, r_i, pl.ds(orr + k * LANES, LANES)]
                        prod = ph * pt * pr
                        acc4 = prod if acc4 is None else acc4 + prod
                    for pm in perms:
                        acc4 = acc4 + shuffle(acc4, pm)
                    acc = jnp.where(lane == rr, acc4, acc)
                out_v[pl.ds(p * ROWS_P + g * LANES, LANES)] = acc
                return 0

            lax.fori_loop(0, GROUPS_P, group, 0)

        pltpu.sync_copy(out_v, out_hbm.at[pl.ds(wid * B_PER_W, B_PER_W)])

    return distmult


_distmult = _make_kernel()


def kernel(batch_h, batch_t, batch_r, ent_embeddings, rel_embeddings):
    bh = batch_h.astype(jnp.int32)
    bt = batch_t.astype(jnp.int32)
    br = batch_r.astype(jnp.int32)
    half = LB // 2
    hp = (((bh // LB) * half) | (bh & (half - 1))).reshape(NW * NCH, CH)
    tp = (((bt // LB) * half) | (bt & (half - 1))).reshape(NW * NCH, CH)
    rp = (br >> 1).reshape(NW * NCH, CH)
    shifts = jnp.arange(LANES, dtype=jnp.int32)
    ph = jnp.sum((((bh // half) & 1).reshape(NW, GROUPS_W, LANES)) << shifts,
                 axis=-1)
    pt = jnp.sum((((bt // half) & 1).reshape(NW, GROUPS_W, LANES)) << shifts,
                 axis=-1)
    pr = jnp.sum(((br & 1).reshape(NW, GROUPS_W, LANES)) << shifts, axis=-1)
    par = jnp.stack([ph, pt, pr], axis=-1).astype(jnp.int32)
    par = jnp.pad(par, ((0, 0), (0, 0), (0, LANES - 3)))
    ent2 = _pair_table(ent_embeddings.T)
    rel2 = rel_embeddings.reshape(REL_P, 2 * DIM)
    return _distmult(hp, tp, rp, par, ent2, rel2)
